# Initial kernel scaffold; baseline (speedup 1.0000x reference)
import jax
import jax.numpy as jnp
from jax.experimental import pallas as pl


def kernel(x, W_enc, m, sd, p):
    raise NotImplementedError("write your pallas kernel here")


# speedup vs baseline: 3.4088x; 3.4088x over previous
import jax
import jax.numpy as jnp
from jax.experimental import pallas as pl
from jax.experimental.pallas import tpu as pltpu

B, C_IN, T, C_LAT, K = 64, 128, 256, 64, 512
D = C_LAT * T
BB = 16
KB = 128
KB2 = 128
_LN_EMA = float(jnp.log(jnp.float32(0.001)))


def _enc_body(wt_ref, x_ref, y_ref):
    wb = wt_ref[...].astype(jnp.bfloat16)
    for i in range(BB):
        xb = x_ref[i].astype(jnp.bfloat16)
        y_ref[i] = jax.lax.dot_general(
            wb, xb, (((1,), (0,)), ((), ())),
            preferred_element_type=jnp.float32)


def _score_body(yf_ref, mf_ref, sT_ref):
    mrow = mf_ref[...]
    mb = mrow.astype(jnp.bfloat16)
    yb = yf_ref[...].astype(jnp.bfloat16)
    dotT = jax.lax.dot_general(
        mb, yb, (((1,), (1,)), ((), ())),
        preferred_element_type=jnp.float32)
    mn = jnp.sum(mrow * mrow, axis=1, keepdims=True)
    sT_ref[...] = mn - 2.0 * dotT


def _weights_body(sT_ref, c_ref, scale_ref):
    sT = sT_ref[...]
    minv = jnp.min(sT, axis=0, keepdims=True)
    kio = jax.lax.broadcasted_iota(jnp.int32, (K, B), 0)
    z = jnp.min(jnp.where(sT == minv, kio, K), axis=0,
                keepdims=True)
    oh = (kio == z)
    ohb = oh.astype(jnp.bfloat16)
    eq = jax.lax.dot_general(ohb, ohb, (((0,), (0,)), ((), ())),
                             preferred_element_type=jnp.float32)
    jio = jax.lax.broadcasted_iota(jnp.int32, (B, B), 0)
    iio = jax.lax.broadcasted_iota(jnp.int32, (B, B), 1)
    later = jnp.sum(jnp.where(jio > iio, eq, 0.0), axis=0,
                    keepdims=True)
    w = 0.999 * jnp.exp(later * _LN_EMA)
    c_ref[...] = oh.astype(jnp.float32) * w
    count = jnp.sum(oh.astype(jnp.float32), axis=1, keepdims=True)
    scale_ref[...] = jnp.exp(count * _LN_EMA)


def _update_body(mf_ref, c_ref, scale_ref, yf_ref, out_ref):
    cb = c_ref[...].astype(jnp.bfloat16)
    yb = yf_ref[...].astype(jnp.bfloat16)
    upd = jax.lax.dot_general(cb, yb, (((1,), (0,)), ((), ())),
                              preferred_element_type=jnp.float32)
    out_ref[...] = scale_ref[...] * mf_ref[...] + upd


def kernel(x, W_enc, m, sd, p):
    del sd, p
    mf = m.reshape(K, D)

    y = pl.pallas_call(
        _enc_body,
        grid=(B // BB,),
        in_specs=[pl.BlockSpec((C_LAT, C_IN), lambda i: (0, 0)),
                  pl.BlockSpec((BB, C_IN, T), lambda i: (i, 0, 0))],
        out_specs=pl.BlockSpec((BB, C_LAT, T), lambda i: (i, 0, 0)),
        out_shape=jax.ShapeDtypeStruct((B, C_LAT, T), jnp.float32),
    )(W_enc.T, x)
    yf = y.reshape(B, D)

    sT = pl.pallas_call(
        _score_body,
        grid=(K // KB,),
        in_specs=[pl.BlockSpec((B, D), lambda i: (0, 0)),
                  pl.BlockSpec((KB, D), lambda i: (i, 0))],
        out_specs=pl.BlockSpec((KB, B), lambda i: (i, 0)),
        out_shape=jax.ShapeDtypeStruct((K, B), jnp.float32),
    )(yf, mf)

    c_kb, scale = pl.pallas_call(
        _weights_body,
        in_specs=[pl.BlockSpec((K, B), lambda: (0, 0))],
        out_specs=[pl.BlockSpec((K, B), lambda: (0, 0)),
                   pl.BlockSpec((K, 1), lambda: (0, 0))],
        out_shape=[jax.ShapeDtypeStruct((K, B), jnp.float32),
                   jax.ShapeDtypeStruct((K, 1), jnp.float32)],
    )(sT)

    out = pl.pallas_call(
        _update_body,
        grid=(K // KB2,),
        in_specs=[pl.BlockSpec((KB2, D), lambda i: (i, 0)),
                  pl.BlockSpec((KB2, B), lambda i: (i, 0)),
                  pl.BlockSpec((KB2, 1), lambda i: (i, 0)),
                  pl.BlockSpec((B, D), lambda i: (0, 0))],
        out_specs=pl.BlockSpec((KB2, D), lambda i: (i, 0)),
        out_shape=jax.ShapeDtypeStruct((K, D), jnp.float32),
    )(mf, c_kb, scale, yf)

    return out.reshape(K, C_LAT, T)


# speedup vs baseline: 4.9589x; 1.4548x over previous
import jax
import jax.numpy as jnp
from jax import lax
from jax.experimental import pallas as pl
from jax.experimental.pallas import tpu as pltpu

B, C_IN, T, C_LAT, K = 64, 128, 256, 64, 512
BB = 16
_LN_EMA = float(jnp.log(jnp.float32(0.001)))
_BF = jnp.bfloat16
_DN_BT = (((1,), (1,)), ((), ()))
_DN_NN = (((1,), (0,)), ((), ()))


def _enc_body(wt_ref, x_ref, y_ref):
    wb = wt_ref[...].astype(_BF)
    for i in range(BB):
        y_ref[i] = lax.dot_general(wb, x_ref[i].astype(_BF), _DN_NN,
                                   preferred_element_type=jnp.float32)


def _mega_body(m_hbm, y_hbm, out_hbm, mscr, yscr, sT, cw, scl, obuf,
               msem, ysem, osem):
    def start_in(j):
        pltpu.make_async_copy(m_hbm.at[:, j, :], mscr.at[j],
                              msem.at[j % 2]).start()
        pltpu.make_async_copy(y_hbm.at[:, j, :], yscr.at[j],
                              ysem.at[j % 2]).start()

    def wait_in(j):
        pltpu.make_async_copy(m_hbm.at[:, j, :], mscr.at[j],
                              msem.at[j % 2]).wait()
        pltpu.make_async_copy(y_hbm.at[:, j, :], yscr.at[j],
                              ysem.at[j % 2]).wait()

    start_in(0)
    start_in(1)

    def phase_a(j, carry):
        wait_in(j)

        @pl.when(j < C_LAT - 2)
        def _():
            start_in(j + 2)

        mj = mscr[j]
        dotT = lax.dot_general(mj.astype(_BF), yscr[j].astype(_BF), _DN_BT,
                               preferred_element_type=jnp.float32)
        inc = jnp.sum(mj * mj, axis=1, keepdims=True) - 2.0 * dotT

        @pl.when(j == 0)
        def _():
            sT[...] = inc

        @pl.when(j > 0)
        def _():
            sT[...] += inc

        return carry

    lax.fori_loop(0, C_LAT, phase_a, 0)

    s = sT[...]
    minv = jnp.min(s, axis=0, keepdims=True)
    kio = lax.broadcasted_iota(jnp.int32, (K, B), 0)
    z = jnp.min(jnp.where(s == minv, kio, K), axis=0, keepdims=True)
    oh = (kio == z)
    ohb = oh.astype(_BF)
    eq = lax.dot_general(ohb, ohb, (((0,), (0,)), ((), ())),
                         preferred_element_type=jnp.float32)
    jio = lax.broadcasted_iota(jnp.int32, (B, B), 0)
    iio = lax.broadcasted_iota(jnp.int32, (B, B), 1)
    later = jnp.sum(jnp.where(jio > iio, eq, 0.0), axis=0, keepdims=True)
    w = 0.999 * jnp.exp(later * _LN_EMA)
    cw[...] = (oh.astype(jnp.float32) * w).astype(_BF)
    count = jnp.sum(oh.astype(jnp.float32), axis=1, keepdims=True)
    scl[...] = jnp.exp(count * _LN_EMA)

    def phase_c(j, carry):
        @pl.when(j >= 2)
        def _():
            pltpu.make_async_copy(obuf.at[j % 2], out_hbm.at[:, j - 2, :],
                                  osem.at[j % 2]).wait()

        upd = lax.dot_general(cw[...], yscr[j].astype(_BF), _DN_NN,
                              preferred_element_type=jnp.float32)
        obuf[j % 2] = scl[...] * mscr[j] + upd
        pltpu.make_async_copy(obuf.at[j % 2], out_hbm.at[:, j, :],
                              osem.at[j % 2]).start()
        return carry

    lax.fori_loop(0, C_LAT, phase_c, 0)
    pltpu.make_async_copy(obuf.at[0], out_hbm.at[:, C_LAT - 2, :],
                          osem.at[0]).wait()
    pltpu.make_async_copy(obuf.at[1], out_hbm.at[:, C_LAT - 1, :],
                          osem.at[1]).wait()


def kernel(x, W_enc, m, sd, p):
    del sd, p

    y = pl.pallas_call(
        _enc_body,
        grid=(B // BB,),
        in_specs=[pl.BlockSpec((C_LAT, C_IN), lambda i: (0, 0)),
                  pl.BlockSpec((BB, C_IN, T), lambda i: (i, 0, 0))],
        out_specs=pl.BlockSpec((BB, C_LAT, T), lambda i: (i, 0, 0)),
        out_shape=jax.ShapeDtypeStruct((B, C_LAT, T), jnp.float32),
    )(W_enc.T, x)

    any_spec = pl.BlockSpec(memory_space=pl.ANY)
    out = pl.pallas_call(
        _mega_body,
        in_specs=[any_spec, any_spec],
        out_specs=any_spec,
        out_shape=jax.ShapeDtypeStruct((K, C_LAT, T), jnp.float32),
        scratch_shapes=[
            pltpu.VMEM((C_LAT, K, T), jnp.float32),
            pltpu.VMEM((C_LAT, B, T), jnp.float32),
            pltpu.VMEM((K, B), jnp.float32),
            pltpu.VMEM((K, B), _BF),
            pltpu.VMEM((K, 1), jnp.float32),
            pltpu.VMEM((2, K, T), jnp.float32),
            pltpu.SemaphoreType.DMA((2,)),
            pltpu.SemaphoreType.DMA((2,)),
            pltpu.SemaphoreType.DMA((2,)),
        ],
    )(m, y)

    return out


# speedup vs baseline: 8.7995x; 1.7745x over previous
import math

import jax
import jax.numpy as jnp
from jax import lax
from jax.experimental import pallas as pl
from jax.experimental.pallas import tpu as pltpu

B, C_IN, T, C_LAT, K = 64, 128, 256, 64, 512
BB = 16
W_IN = 8
W_OUT = 4
_LN_EMA = math.log(0.001)
_BF = jnp.bfloat16
_DN_BT = (((1,), (1,)), ((), ()))
_DN_NN = (((1,), (0,)), ((), ()))


def _enc_body(wt_ref, x_ref, y_ref):
    wb = wt_ref[...].astype(_BF)
    for i in range(BB):
        y_ref[i] = lax.dot_general(wb, x_ref[i].astype(_BF), _DN_NN,
                                   preferred_element_type=jnp.float32)


def _mega_body(m_hbm, y_hbm, out_hbm, mscr, yscr, sT, cw, scl, obuf,
               msem, ysem, osem):
    def start_in(j):
        pltpu.make_async_copy(m_hbm.at[:, j, :], mscr.at[j],
                              msem.at[j % W_IN]).start()
        pltpu.make_async_copy(y_hbm.at[:, j, :], yscr.at[j],
                              ysem.at[j % W_IN]).start()

    def wait_in(j):
        pltpu.make_async_copy(m_hbm.at[:, j, :], mscr.at[j],
                              msem.at[j % W_IN]).wait()
        pltpu.make_async_copy(y_hbm.at[:, j, :], yscr.at[j],
                              ysem.at[j % W_IN]).wait()

    for jj in range(W_IN):
        start_in(jj)

    def phase_a(j, carry):
        wait_in(j)

        @pl.when(j < C_LAT - W_IN)
        def _():
            start_in(j + W_IN)

        mj = mscr[j]
        dotT = lax.dot_general(mj.astype(_BF), yscr[j].astype(_BF), _DN_BT,
                               preferred_element_type=jnp.float32)
        inc = jnp.sum(mj * mj, axis=1, keepdims=True) - 2.0 * dotT

        @pl.when(j == 0)
        def _():
            sT[...] = inc

        @pl.when(j > 0)
        def _():
            sT[...] += inc

        return carry

    lax.fori_loop(0, C_LAT, phase_a, 0)

    s = sT[...]
    minv = jnp.min(s, axis=0, keepdims=True)
    kio = lax.broadcasted_iota(jnp.int32, (K, B), 0)
    z = jnp.min(jnp.where(s == minv, kio, K), axis=0, keepdims=True)
    oh = (kio == z)
    ohb = oh.astype(_BF)
    eq = lax.dot_general(ohb, ohb, (((0,), (0,)), ((), ())),
                         preferred_element_type=jnp.float32)
    jio = lax.broadcasted_iota(jnp.int32, (B, B), 0)
    iio = lax.broadcasted_iota(jnp.int32, (B, B), 1)
    later = jnp.sum(jnp.where(jio > iio, eq, 0.0), axis=0, keepdims=True)
    w = 0.999 * jnp.exp(later * _LN_EMA)
    cw[...] = (oh.astype(jnp.float32) * w).astype(_BF)
    count = jnp.sum(oh.astype(jnp.float32), axis=1, keepdims=True)
    scl[...] = jnp.exp(count * _LN_EMA)

    def phase_c(j, carry):
        @pl.when(j >= W_OUT)
        def _():
            pltpu.make_async_copy(obuf.at[j % W_OUT],
                                  out_hbm.at[:, j - W_OUT, :],
                                  osem.at[j % W_OUT]).wait()

        upd = lax.dot_general(cw[...], yscr[j].astype(_BF), _DN_NN,
                              preferred_element_type=jnp.float32)
        obuf[j % W_OUT] = scl[...] * mscr[j] + upd
        pltpu.make_async_copy(obuf.at[j % W_OUT], out_hbm.at[:, j, :],
                              osem.at[j % W_OUT]).start()
        return carry

    lax.fori_loop(0, C_LAT, phase_c, 0)
    for jj in range(C_LAT - W_OUT, C_LAT):
        pltpu.make_async_copy(obuf.at[jj % W_OUT], out_hbm.at[:, jj, :],
                              osem.at[jj % W_OUT]).wait()


def kernel(x, W_enc, m, sd, p):
    del sd, p

    y = pl.pallas_call(
        _enc_body,
        grid=(B // BB,),
        in_specs=[pl.BlockSpec((C_LAT, C_IN), lambda i: (0, 0)),
                  pl.BlockSpec((BB, C_IN, T), lambda i: (i, 0, 0))],
        out_specs=pl.BlockSpec((BB, C_LAT, T), lambda i: (i, 0, 0)),
        out_shape=jax.ShapeDtypeStruct((B, C_LAT, T), jnp.float32),
    )(W_enc.T, x)

    any_spec = pl.BlockSpec(memory_space=pl.ANY)
    out = pl.pallas_call(
        _mega_body,
        in_specs=[any_spec, any_spec],
        out_specs=any_spec,
        out_shape=jax.ShapeDtypeStruct((K, C_LAT, T), jnp.float32),
        scratch_shapes=[
            pltpu.VMEM((C_LAT, K, T), jnp.float32),
            pltpu.VMEM((C_LAT, B, T), jnp.float32),
            pltpu.VMEM((K, B), jnp.float32),
            pltpu.VMEM((K, B), _BF),
            pltpu.VMEM((K, 1), jnp.float32),
            pltpu.VMEM((W_OUT, K, T), jnp.float32),
            pltpu.SemaphoreType.DMA((W_IN,)),
            pltpu.SemaphoreType.DMA((W_IN,)),
            pltpu.SemaphoreType.DMA((W_OUT,)),
        ],
    )(m, y)

    return out


# speedup vs baseline: 9.2961x; 1.0564x over previous
import math

import jax
import jax.numpy as jnp
from jax import lax
from jax.experimental import pallas as pl
from jax.experimental.pallas import tpu as pltpu

B, C_IN, T, C_LAT, K = 64, 128, 256, 64, 512
BB = 16
W_IN = 16
W_OUT = 8
_LN_EMA = math.log(0.001)
_BF = jnp.bfloat16
_DN_BT = (((1,), (1,)), ((), ()))
_DN_NN = (((1,), (0,)), ((), ()))


def _enc_body(wt_ref, x_ref, y_ref):
    wb = wt_ref[...].astype(_BF)
    for i in range(BB):
        y_ref[i] = lax.dot_general(wb, x_ref[i].astype(_BF), _DN_NN,
                                   preferred_element_type=jnp.float32)


def _mega_body(m_hbm, y_hbm, out_hbm, mscr, yscr, sT, cw, scl, obuf,
               msem, ysem, osem):
    def start_in(j):
        pltpu.make_async_copy(m_hbm.at[:, j, :], mscr.at[j],
                              msem.at[j % W_IN]).start()
        pltpu.make_async_copy(y_hbm.at[:, j, :], yscr.at[j],
                              ysem.at[j % W_IN]).start()

    def wait_in(j):
        pltpu.make_async_copy(m_hbm.at[:, j, :], mscr.at[j],
                              msem.at[j % W_IN]).wait()
        pltpu.make_async_copy(y_hbm.at[:, j, :], yscr.at[j],
                              ysem.at[j % W_IN]).wait()

    for jj in range(W_IN):
        start_in(jj)

    def phase_a(j, carry):
        wait_in(j)

        @pl.when(j < C_LAT - W_IN)
        def _():
            start_in(j + W_IN)

        mj = mscr[j]
        dotT = lax.dot_general(mj.astype(_BF), yscr[j].astype(_BF), _DN_BT,
                               preferred_element_type=jnp.float32)
        inc = jnp.sum(mj * mj, axis=1, keepdims=True) - 2.0 * dotT

        @pl.when(j == 0)
        def _():
            sT[...] = inc

        @pl.when(j > 0)
        def _():
            sT[...] += inc

        return carry

    lax.fori_loop(0, C_LAT, phase_a, 0)

    s = sT[...]
    minv = jnp.min(s, axis=0, keepdims=True)
    kio = lax.broadcasted_iota(jnp.int32, (K, B), 0)
    z = jnp.min(jnp.where(s == minv, kio, K), axis=0, keepdims=True)
    oh = (kio == z)
    ohb = oh.astype(_BF)
    eq = lax.dot_general(ohb, ohb, (((0,), (0,)), ((), ())),
                         preferred_element_type=jnp.float32)
    jio = lax.broadcasted_iota(jnp.int32, (B, B), 0)
    iio = lax.broadcasted_iota(jnp.int32, (B, B), 1)
    later = jnp.sum(jnp.where(jio > iio, eq, 0.0), axis=0, keepdims=True)
    w = 0.999 * jnp.exp(later * _LN_EMA)
    cw[...] = (oh.astype(jnp.float32) * w).astype(_BF)
    count = jnp.sum(oh.astype(jnp.float32), axis=1, keepdims=True)
    scl[...] = jnp.exp(count * _LN_EMA)

    def phase_c(j, carry):
        @pl.when(j >= W_OUT)
        def _():
            pltpu.make_async_copy(obuf.at[j % W_OUT],
                                  out_hbm.at[:, j - W_OUT, :],
                                  osem.at[j % W_OUT]).wait()

        upd = lax.dot_general(cw[...], yscr[j].astype(_BF), _DN_NN,
                              preferred_element_type=jnp.float32)
        obuf[j % W_OUT] = scl[...] * mscr[j] + upd
        pltpu.make_async_copy(obuf.at[j % W_OUT], out_hbm.at[:, j, :],
                              osem.at[j % W_OUT]).start()
        return carry

    lax.fori_loop(0, C_LAT, phase_c, 0)
    for jj in range(C_LAT - W_OUT, C_LAT):
        pltpu.make_async_copy(obuf.at[jj % W_OUT], out_hbm.at[:, jj, :],
                              osem.at[jj % W_OUT]).wait()


def kernel(x, W_enc, m, sd, p):
    del sd, p

    y = pl.pallas_call(
        _enc_body,
        grid=(B // BB,),
        in_specs=[pl.BlockSpec((C_LAT, C_IN), lambda i: (0, 0)),
                  pl.BlockSpec((BB, C_IN, T), lambda i: (i, 0, 0))],
        out_specs=pl.BlockSpec((BB, C_LAT, T), lambda i: (i, 0, 0)),
        out_shape=jax.ShapeDtypeStruct((B, C_LAT, T), jnp.float32),
    )(W_enc.T, x)

    any_spec = pl.BlockSpec(memory_space=pl.ANY)
    out = pl.pallas_call(
        _mega_body,
        in_specs=[any_spec, any_spec],
        out_specs=any_spec,
        out_shape=jax.ShapeDtypeStruct((K, C_LAT, T), jnp.float32),
        scratch_shapes=[
            pltpu.VMEM((C_LAT, K, T), jnp.float32),
            pltpu.VMEM((C_LAT, B, T), jnp.float32),
            pltpu.VMEM((K, B), jnp.float32),
            pltpu.VMEM((K, B), _BF),
            pltpu.VMEM((K, 1), jnp.float32),
            pltpu.VMEM((W_OUT, K, T), jnp.float32),
            pltpu.SemaphoreType.DMA((W_IN,)),
            pltpu.SemaphoreType.DMA((W_IN,)),
            pltpu.SemaphoreType.DMA((W_OUT,)),
        ],
    )(m, y)

    return out
